# baseline (speedup 1.0000x reference)
import functools
import math

import jax
import jax.numpy as jnp
from jax import lax
from jax.experimental import pallas as pl
from jax.experimental.pallas import tpu as pltpu
from jax.experimental.pallas import tpu_sc as plsc

_CUTOFF = 5.0
_NC = 2
_NS = 16
_NW = _NC * _NS

_SUB = 80
_NSUB = 50
_G = _SUB * _NSUB
_WIN = 8


def _switch_poly(t):
    half = t * 0.5
    y = half * half
    c = -1.0 / 3628800.0
    c = c * y + (1.0 / 40320.0)
    c = c * y + (-1.0 / 720.0)
    c = c * y + (1.0 / 24.0)
    c = c * y + (-0.5)
    c = c * y + 1.0
    return c * c


def _body(n_nodes, rows_per_w, ngroups,
          tbl_hbm, src_hbm, fidx_hbm,
          o4_out, sw_out, mask_out,
          idx_v, src_v, r4_v, sw_v, mask_v,
          sem_g, sem_out):
    cid = lax.axis_index("c")
    sid = lax.axis_index("s")
    wid = sid * _NC + cid
    base_row = wid * rows_per_w

    k = math.pi / _CUTOFF
    lane = jax.lax.iota(jnp.int32, 16)

    def group(g, carry):
        row0 = base_row + g * _NSUB
        pltpu.sync_copy(fidx_hbm.at[pl.ds(row0, _NSUB)], idx_v)
        pltpu.sync_copy(src_hbm.at[pl.ds(row0, _NSUB)], src_v)

        def step(j, c2):
            @pl.when(j < _NSUB)
            def _fire():
                pltpu.async_copy(tbl_hbm.at[idx_v.at[j]], r4_v.at[j], sem_g)

            @pl.when(j >= _WIN)
            def _drain():
                jj = j - _WIN
                pltpu.make_async_copy(
                    tbl_hbm.at[idx_v.at[jj]], r4_v.at[jj], sem_g).wait()
            return c2

        lax.fori_loop(0, _NSUB + _WIN, step, 0)

        def compute(j, c3):
            j16 = jnp.full((16,), 0, jnp.int32) + j
            col3 = jnp.full((16,), 3, jnp.int32)
            for kk in range(_SUB // 16):
                sl = pl.ds(kk * 16, 16)
                d = plsc.load_gather(r4_v, [j16, kk * 16 + lane, col3])
                s = src_v[j, sl]
                m = s < n_nodes
                sw = _switch_poly(d * k)
                sw_v[j, sl] = jnp.where(m, sw, 0.0)
                mask_v[j, sl] = jnp.where(m, 1, 0)
            return c3

        lax.fori_loop(0, _NSUB, compute, 0)

        out_sl = pl.ds(row0, _NSUB)
        pltpu.async_copy(r4_v, o4_out.at[out_sl], sem_out)
        pltpu.async_copy(sw_v, sw_out.at[out_sl], sem_out)
        pltpu.async_copy(mask_v, mask_out.at[out_sl], sem_out)
        pltpu.make_async_copy(r4_v, o4_out.at[out_sl], sem_out).wait()
        pltpu.make_async_copy(sw_v, sw_out.at[out_sl], sem_out).wait()
        pltpu.make_async_copy(mask_v, mask_out.at[out_sl], sem_out).wait()
        return carry

    lax.fori_loop(0, ngroups, group, 0)


def kernel(coordinates, vec, distances, edge_src, filter_indices):
    n_nodes = coordinates.shape[0]
    e_out = edge_src.shape[0]
    assert e_out % (_NW * _G) == 0
    nrows = e_out // _SUB
    rows_per_w = nrows // _NW
    ngroups = rows_per_w // _NSUB

    fidx2 = filter_indices.reshape(nrows, _SUB)
    src2 = edge_src.reshape(nrows, _SUB)
    e_in = vec.shape[0]
    tbl = jnp.concatenate(
        [vec, distances[:, None], jnp.zeros((e_in, 4), jnp.float32)], axis=1)

    mesh = plsc.VectorSubcoreMesh(
        core_axis_name="c", subcore_axis_name="s",
        num_cores=_NC, num_subcores=_NS)
    run = functools.partial(
        pl.kernel,
        mesh=mesh,
        compiler_params=pltpu.CompilerParams(
            use_tc_tiling_on_sc=False, needs_layout_passes=False),
        out_type=[
            jax.ShapeDtypeStruct((nrows, _SUB, 8), jnp.float32),
            jax.ShapeDtypeStruct((nrows, _SUB), jnp.float32),
            jax.ShapeDtypeStruct((nrows, _SUB), jnp.int32),
        ],
        scratch_types=[
            pltpu.VMEM((_NSUB, _SUB), jnp.int32),
            pltpu.VMEM((_NSUB, _SUB), jnp.int32),
            pltpu.VMEM((_NSUB, _SUB, 8), jnp.float32),
            pltpu.VMEM((_NSUB, _SUB), jnp.float32),
            pltpu.VMEM((_NSUB, _SUB), jnp.int32),
            pltpu.SemaphoreType.DMA,
            pltpu.SemaphoreType.DMA,
        ],
    )(functools.partial(_body, n_nodes, rows_per_w, ngroups))

    o8, switch, mask_i32 = run(tbl, src2, fidx2)
    o8r = o8.reshape(e_out, 8)
    return (o8r[:, :3], o8r[:, 3],
            switch.reshape(e_out), mask_i32.reshape(e_out).astype(jnp.bool_))


# speedup vs baseline: 8.0933x; 8.0933x over previous
import functools
import math

import jax
import jax.numpy as jnp
from jax import lax
from jax.experimental import pallas as pl
from jax.experimental.pallas import tpu as pltpu
from jax.experimental.pallas import tpu_sc as plsc

_CUTOFF = 5.0
_NC = 2
_NS = 16
_NW = _NC * _NS

_SUB = 800
_NSUB = 5
_G = _SUB * _NSUB
_WIN = 2


def _switch_poly(t):
    half = t * 0.5
    y = half * half
    c = -1.0 / 3628800.0
    c = c * y + (1.0 / 40320.0)
    c = c * y + (-1.0 / 720.0)
    c = c * y + (1.0 / 24.0)
    c = c * y + (-0.5)
    c = c * y + 1.0
    return c * c


def _body(n_nodes, rows_per_w, ngroups,
          vx_hbm, vy_hbm, vz_hbm, dist_hbm, src_hbm, fidx_hbm,
          ox_out, oy_out, oz_out, dist_out, sw_out, mask_out,
          idx_v, src_v, rx_v, ry_v, rz_v, dist_v, sw_v, mask_v,
          sem_vec, sem_dist, sem_out):
    cid = lax.axis_index("c")
    sid = lax.axis_index("s")
    wid = sid * _NC + cid
    base_row = wid * rows_per_w

    k = math.pi / _CUTOFF

    def group(g, carry):
        row0 = base_row + g * _NSUB
        pltpu.sync_copy(fidx_hbm.at[pl.ds(row0, _NSUB)], idx_v)
        pltpu.sync_copy(src_hbm.at[pl.ds(row0, _NSUB)], src_v)

        def step(j, c2):
            @pl.when(j < _NSUB)
            def _fire():
                pltpu.async_copy(vx_hbm.at[idx_v.at[j]], rx_v.at[j], sem_vec)
                pltpu.async_copy(vy_hbm.at[idx_v.at[j]], ry_v.at[j], sem_vec)
                pltpu.async_copy(vz_hbm.at[idx_v.at[j]], rz_v.at[j], sem_vec)
                pltpu.async_copy(dist_hbm.at[idx_v.at[j]], dist_v.at[j], sem_dist)

            @pl.when(j >= _WIN)
            def _drain():
                jj = j - _WIN
                pltpu.make_async_copy(
                    vx_hbm.at[idx_v.at[jj]], rx_v.at[jj], sem_vec).wait()
                pltpu.make_async_copy(
                    vy_hbm.at[idx_v.at[jj]], ry_v.at[jj], sem_vec).wait()
                pltpu.make_async_copy(
                    vz_hbm.at[idx_v.at[jj]], rz_v.at[jj], sem_vec).wait()
                pltpu.make_async_copy(
                    dist_hbm.at[idx_v.at[jj]], dist_v.at[jj], sem_dist).wait()
            return c2

        lax.fori_loop(0, _NSUB + _WIN, step, 0)

        def compute(j, c3):
            for kk in range(_SUB // 16):
                sl = pl.ds(kk * 16, 16)
                d = dist_v[j, sl]
                s = src_v[j, sl]
                m = s < n_nodes
                sw = _switch_poly(d * k)
                sw_v[j, sl] = jnp.where(m, sw, 0.0)
                mask_v[j, sl] = jnp.where(m, 1, 0)
            return c3

        lax.fori_loop(0, _NSUB, compute, 0)

        out_sl = pl.ds(row0, _NSUB)
        pltpu.async_copy(rx_v, ox_out.at[out_sl], sem_out)
        pltpu.async_copy(ry_v, oy_out.at[out_sl], sem_out)
        pltpu.async_copy(rz_v, oz_out.at[out_sl], sem_out)
        pltpu.async_copy(dist_v, dist_out.at[out_sl], sem_out)
        pltpu.async_copy(sw_v, sw_out.at[out_sl], sem_out)
        pltpu.async_copy(mask_v, mask_out.at[out_sl], sem_out)
        pltpu.make_async_copy(rx_v, ox_out.at[out_sl], sem_out).wait()
        pltpu.make_async_copy(ry_v, oy_out.at[out_sl], sem_out).wait()
        pltpu.make_async_copy(rz_v, oz_out.at[out_sl], sem_out).wait()
        pltpu.make_async_copy(dist_v, dist_out.at[out_sl], sem_out).wait()
        pltpu.make_async_copy(sw_v, sw_out.at[out_sl], sem_out).wait()
        pltpu.make_async_copy(mask_v, mask_out.at[out_sl], sem_out).wait()
        return carry

    lax.fori_loop(0, ngroups, group, 0)


def kernel(coordinates, vec, distances, edge_src, filter_indices):
    n_nodes = coordinates.shape[0]
    e_out = edge_src.shape[0]
    assert e_out % (_NW * _G) == 0
    nrows = e_out // _SUB
    rows_per_w = nrows // _NW
    ngroups = rows_per_w // _NSUB

    fidx2 = filter_indices.reshape(nrows, _SUB)
    src2 = edge_src.reshape(nrows, _SUB)
    vx = vec[:, 0]
    vy = vec[:, 1]
    vz = vec[:, 2]

    mesh = plsc.VectorSubcoreMesh(
        core_axis_name="c", subcore_axis_name="s",
        num_cores=_NC, num_subcores=_NS)
    run = functools.partial(
        pl.kernel,
        mesh=mesh,
        compiler_params=pltpu.CompilerParams(use_tc_tiling_on_sc=False),
        out_type=[
            jax.ShapeDtypeStruct((nrows, _SUB), jnp.float32),
            jax.ShapeDtypeStruct((nrows, _SUB), jnp.float32),
            jax.ShapeDtypeStruct((nrows, _SUB), jnp.float32),
            jax.ShapeDtypeStruct((nrows, _SUB), jnp.float32),
            jax.ShapeDtypeStruct((nrows, _SUB), jnp.float32),
            jax.ShapeDtypeStruct((nrows, _SUB), jnp.int32),
        ],
        scratch_types=[
            pltpu.VMEM((_NSUB, _SUB), jnp.int32),
            pltpu.VMEM((_NSUB, _SUB), jnp.int32),
            pltpu.VMEM((_NSUB, _SUB), jnp.float32),
            pltpu.VMEM((_NSUB, _SUB), jnp.float32),
            pltpu.VMEM((_NSUB, _SUB), jnp.float32),
            pltpu.VMEM((_NSUB, _SUB), jnp.float32),
            pltpu.VMEM((_NSUB, _SUB), jnp.float32),
            pltpu.VMEM((_NSUB, _SUB), jnp.int32),
            pltpu.SemaphoreType.DMA,
            pltpu.SemaphoreType.DMA,
            pltpu.SemaphoreType.DMA,
        ],
    )(functools.partial(_body, n_nodes, rows_per_w, ngroups))

    ox, oy, oz, dist_g, switch, mask_i32 = run(vx, vy, vz, distances, src2, fidx2)
    vec_g = jnp.stack([ox, oy, oz], axis=-1).reshape(e_out, 3)
    return (vec_g, dist_g.reshape(e_out),
            switch.reshape(e_out), mask_i32.reshape(e_out).astype(jnp.bool_))


# speedup vs baseline: 8.4134x; 1.0395x over previous
import functools
import math

import jax
import jax.numpy as jnp
from jax import lax
from jax.experimental import pallas as pl
from jax.experimental.pallas import tpu as pltpu
from jax.experimental.pallas import tpu_sc as plsc

_CUTOFF = 5.0
_NC = 2
_NS = 16
_NW = _NC * _NS

_SUB = 800
_NSUB = 5
_G = _SUB * _NSUB
_WIN = 2


def _switch_poly(t):
    half = t * 0.5
    y = half * half
    c = -1.0 / 3628800.0
    c = c * y + (1.0 / 40320.0)
    c = c * y + (-1.0 / 720.0)
    c = c * y + (1.0 / 24.0)
    c = c * y + (-0.5)
    c = c * y + 1.0
    return c * c


def _body(n_nodes, per_w, ngroups,
          vx_hbm, vy_hbm, vz_hbm, dist_hbm, src_hbm, fidx_hbm,
          ox_out, oy_out, oz_out, dist_out, sw_out, mask_out,
          idx_v, src_v, rx_v, ry_v, rz_v, dist_v, sw_v, mask_v,
          sem_vec, sem_dist, sem_out):
    cid = lax.axis_index("c")
    sid = lax.axis_index("s")
    wid = sid * _NC + cid
    base = wid * per_w

    k = math.pi / _CUTOFF

    def group(g, carry):
        off = base + g * _G
        pltpu.sync_copy(fidx_hbm.at[pl.ds(off, _G)], idx_v)
        pltpu.sync_copy(src_hbm.at[pl.ds(off, _G)], src_v)

        def step(j, c2):
            @pl.when(j < _NSUB)
            def _fire():
                sl = pl.ds(j * _SUB, _SUB)
                pltpu.async_copy(vx_hbm.at[idx_v.at[sl]], rx_v.at[sl], sem_vec)
                pltpu.async_copy(vy_hbm.at[idx_v.at[sl]], ry_v.at[sl], sem_vec)
                pltpu.async_copy(vz_hbm.at[idx_v.at[sl]], rz_v.at[sl], sem_vec)
                pltpu.async_copy(dist_hbm.at[idx_v.at[sl]], dist_v.at[sl], sem_dist)

            @pl.when(j >= _WIN)
            def _drain():
                sl = pl.ds((j - _WIN) * _SUB, _SUB)
                pltpu.make_async_copy(
                    vx_hbm.at[idx_v.at[sl]], rx_v.at[sl], sem_vec).wait()
                pltpu.make_async_copy(
                    vy_hbm.at[idx_v.at[sl]], ry_v.at[sl], sem_vec).wait()
                pltpu.make_async_copy(
                    vz_hbm.at[idx_v.at[sl]], rz_v.at[sl], sem_vec).wait()
                pltpu.make_async_copy(
                    dist_hbm.at[idx_v.at[sl]], dist_v.at[sl], sem_dist).wait()
            return c2

        lax.fori_loop(0, _NSUB + _WIN, step, 0)

        def compute(i, c3):
            sl = pl.ds(i * 16, 16)
            d = dist_v[sl]
            s = src_v[sl]
            m = s < n_nodes
            sw = _switch_poly(d * k)
            sw_v[sl] = jnp.where(m, sw, 0.0)
            mask_v[sl] = jnp.where(m, 1, 0)
            return c3

        lax.fori_loop(0, _G // 16, compute, 0)

        out_sl = pl.ds(off, _G)
        pltpu.async_copy(rx_v, ox_out.at[out_sl], sem_out)
        pltpu.async_copy(ry_v, oy_out.at[out_sl], sem_out)
        pltpu.async_copy(rz_v, oz_out.at[out_sl], sem_out)
        pltpu.async_copy(dist_v, dist_out.at[out_sl], sem_out)
        pltpu.async_copy(sw_v, sw_out.at[out_sl], sem_out)
        pltpu.async_copy(mask_v, mask_out.at[out_sl], sem_out)
        pltpu.make_async_copy(rx_v, ox_out.at[out_sl], sem_out).wait()
        pltpu.make_async_copy(ry_v, oy_out.at[out_sl], sem_out).wait()
        pltpu.make_async_copy(rz_v, oz_out.at[out_sl], sem_out).wait()
        pltpu.make_async_copy(dist_v, dist_out.at[out_sl], sem_out).wait()
        pltpu.make_async_copy(sw_v, sw_out.at[out_sl], sem_out).wait()
        pltpu.make_async_copy(mask_v, mask_out.at[out_sl], sem_out).wait()
        return carry

    lax.fori_loop(0, ngroups, group, 0)


def kernel(coordinates, vec, distances, edge_src, filter_indices):
    n_nodes = coordinates.shape[0]
    e_out = edge_src.shape[0]
    assert e_out % (_NW * _G) == 0
    per_w = e_out // _NW
    ngroups = per_w // _G

    vx = vec[:, 0]
    vy = vec[:, 1]
    vz = vec[:, 2]

    mesh = plsc.VectorSubcoreMesh(
        core_axis_name="c", subcore_axis_name="s",
        num_cores=_NC, num_subcores=_NS)
    run = functools.partial(
        pl.kernel,
        mesh=mesh,
        compiler_params=pltpu.CompilerParams(use_tc_tiling_on_sc=False),
        out_type=[
            jax.ShapeDtypeStruct((e_out,), jnp.float32),
            jax.ShapeDtypeStruct((e_out,), jnp.float32),
            jax.ShapeDtypeStruct((e_out,), jnp.float32),
            jax.ShapeDtypeStruct((e_out,), jnp.float32),
            jax.ShapeDtypeStruct((e_out,), jnp.float32),
            jax.ShapeDtypeStruct((e_out,), jnp.int32),
        ],
        scratch_types=[
            pltpu.VMEM((_G,), jnp.int32),
            pltpu.VMEM((_G,), jnp.int32),
            pltpu.VMEM((_G,), jnp.float32),
            pltpu.VMEM((_G,), jnp.float32),
            pltpu.VMEM((_G,), jnp.float32),
            pltpu.VMEM((_G,), jnp.float32),
            pltpu.VMEM((_G,), jnp.float32),
            pltpu.VMEM((_G,), jnp.int32),
            pltpu.SemaphoreType.DMA,
            pltpu.SemaphoreType.DMA,
            pltpu.SemaphoreType.DMA,
        ],
    )(functools.partial(_body, n_nodes, per_w, ngroups))

    ox, oy, oz, dist_g, switch, mask_i32 = run(
        vx, vy, vz, distances, edge_src, filter_indices)
    vec_g = jnp.stack([ox, oy, oz], axis=-1)
    return vec_g, dist_g, switch, mask_i32.astype(jnp.bool_)


# speedup vs baseline: 8.8131x; 1.0475x over previous
import functools
import math

import jax
import jax.numpy as jnp
from jax import lax
from jax.experimental import pallas as pl
from jax.experimental.pallas import tpu as pltpu
from jax.experimental.pallas import tpu_sc as plsc

_CUTOFF = 5.0
_NC = 2
_NS = 16
_NW = _NC * _NS

_SUB = 1000
_NSUB = 10
_G = _SUB * _NSUB
_WIN = 3


def _switch_poly(t):
    half = t * 0.5
    y = half * half
    c = -1.0 / 3628800.0
    c = c * y + (1.0 / 40320.0)
    c = c * y + (-1.0 / 720.0)
    c = c * y + (1.0 / 24.0)
    c = c * y + (-0.5)
    c = c * y + 1.0
    return c * c


def _body(n_nodes, per_w, ngroups,
          vx_hbm, vy_hbm, vz_hbm, dist_hbm, src_hbm, fidx_hbm,
          ox_out, oy_out, oz_out, dist_out, sw_out, mask_out,
          idx_v, src_v, rx_v, ry_v, rz_v, dist_v, sw_v, mask_v,
          sem_vec, sem_dist, sem_out):
    cid = lax.axis_index("c")
    sid = lax.axis_index("s")
    wid = sid * _NC + cid
    base = wid * per_w

    k = math.pi / _CUTOFF

    def _wait_outputs(off):
        out_sl = pl.ds(off, _G)
        pltpu.make_async_copy(rx_v, ox_out.at[out_sl], sem_out).wait()
        pltpu.make_async_copy(ry_v, oy_out.at[out_sl], sem_out).wait()
        pltpu.make_async_copy(rz_v, oz_out.at[out_sl], sem_out).wait()
        pltpu.make_async_copy(dist_v, dist_out.at[out_sl], sem_out).wait()
        pltpu.make_async_copy(sw_v, sw_out.at[out_sl], sem_out).wait()
        pltpu.make_async_copy(mask_v, mask_out.at[out_sl], sem_out).wait()

    def group(g, carry):
        off = base + g * _G
        pltpu.sync_copy(fidx_hbm.at[pl.ds(off, _G)], idx_v)
        pltpu.sync_copy(src_hbm.at[pl.ds(off, _G)], src_v)

        @pl.when(g > 0)
        def _prev():
            _wait_outputs(off - _G)

        def step(j, c2):
            @pl.when(j < _NSUB)
            def _fire():
                sl = pl.ds(j * _SUB, _SUB)
                pltpu.async_copy(vx_hbm.at[idx_v.at[sl]], rx_v.at[sl], sem_vec)
                pltpu.async_copy(vy_hbm.at[idx_v.at[sl]], ry_v.at[sl], sem_vec)
                pltpu.async_copy(vz_hbm.at[idx_v.at[sl]], rz_v.at[sl], sem_vec)
                pltpu.async_copy(dist_hbm.at[idx_v.at[sl]], dist_v.at[sl], sem_dist)

            @pl.when(j >= _WIN)
            def _drain():
                sl = pl.ds((j - _WIN) * _SUB, _SUB)
                pltpu.make_async_copy(
                    vx_hbm.at[idx_v.at[sl]], rx_v.at[sl], sem_vec).wait()
                pltpu.make_async_copy(
                    vy_hbm.at[idx_v.at[sl]], ry_v.at[sl], sem_vec).wait()
                pltpu.make_async_copy(
                    vz_hbm.at[idx_v.at[sl]], rz_v.at[sl], sem_vec).wait()
                pltpu.make_async_copy(
                    dist_hbm.at[idx_v.at[sl]], dist_v.at[sl], sem_dist).wait()
            return c2

        lax.fori_loop(0, _NSUB + _WIN, step, 0)

        def compute(i, c3):
            sl = pl.ds(i * 16, 16)
            d = dist_v[sl]
            s = src_v[sl]
            m = s < n_nodes
            sw = _switch_poly(d * k)
            sw_v[sl] = jnp.where(m, sw, 0.0)
            mask_v[sl] = jnp.where(m, 1, 0)
            return c3

        lax.fori_loop(0, _G // 16, compute, 0)

        out_sl = pl.ds(off, _G)
        pltpu.async_copy(rx_v, ox_out.at[out_sl], sem_out)
        pltpu.async_copy(ry_v, oy_out.at[out_sl], sem_out)
        pltpu.async_copy(rz_v, oz_out.at[out_sl], sem_out)
        pltpu.async_copy(dist_v, dist_out.at[out_sl], sem_out)
        pltpu.async_copy(sw_v, sw_out.at[out_sl], sem_out)
        pltpu.async_copy(mask_v, mask_out.at[out_sl], sem_out)
        return carry

    lax.fori_loop(0, ngroups, group, 0)
    _wait_outputs(base + (ngroups - 1) * _G)


def kernel(coordinates, vec, distances, edge_src, filter_indices):
    n_nodes = coordinates.shape[0]
    e_out = edge_src.shape[0]
    assert e_out % (_NW * _G) == 0
    per_w = e_out // _NW
    ngroups = per_w // _G

    vx = vec[:, 0]
    vy = vec[:, 1]
    vz = vec[:, 2]

    mesh = plsc.VectorSubcoreMesh(
        core_axis_name="c", subcore_axis_name="s",
        num_cores=_NC, num_subcores=_NS)
    run = functools.partial(
        pl.kernel,
        mesh=mesh,
        compiler_params=pltpu.CompilerParams(use_tc_tiling_on_sc=False),
        out_type=[
            jax.ShapeDtypeStruct((e_out,), jnp.float32),
            jax.ShapeDtypeStruct((e_out,), jnp.float32),
            jax.ShapeDtypeStruct((e_out,), jnp.float32),
            jax.ShapeDtypeStruct((e_out,), jnp.float32),
            jax.ShapeDtypeStruct((e_out,), jnp.float32),
            jax.ShapeDtypeStruct((e_out,), jnp.int32),
        ],
        scratch_types=[
            pltpu.VMEM((_G,), jnp.int32),
            pltpu.VMEM((_G,), jnp.int32),
            pltpu.VMEM((_G,), jnp.float32),
            pltpu.VMEM((_G,), jnp.float32),
            pltpu.VMEM((_G,), jnp.float32),
            pltpu.VMEM((_G,), jnp.float32),
            pltpu.VMEM((_G,), jnp.float32),
            pltpu.VMEM((_G,), jnp.int32),
            pltpu.SemaphoreType.DMA,
            pltpu.SemaphoreType.DMA,
            pltpu.SemaphoreType.DMA,
        ],
    )(functools.partial(_body, n_nodes, per_w, ngroups))

    ox, oy, oz, dist_g, switch, mask_i32 = run(
        vx, vy, vz, distances, edge_src, filter_indices)
    vec_g = jnp.stack([ox, oy, oz], axis=-1)
    return vec_g, dist_g, switch, mask_i32.astype(jnp.bool_)
